# baseline (speedup 1.0000x reference)
import dataclasses
import functools

import jax
import jax.numpy as jnp
from jax import lax
from jax.experimental import pallas as pl
from jax.experimental.pallas import tpu as pltpu
from jax.experimental.pallas import tpu_sc as plsc

_NUM_SC_CORES = 2
_NUM_SC_SUBCORES = 16
_SC_LANES = 16


def _sc_present(gs_flat, rowoff, B, V, L):
    NW = _NUM_SC_CORES * _NUM_SC_SUBCORES
    RP = B // NW
    CH = RP * V
    NI = RP * L
    mesh = plsc.VectorSubcoreMesh(core_axis_name="c", subcore_axis_name="s")
    cp = pltpu.CompilerParams()
    if "needs_layout_passes" in pltpu.CompilerParams.__dataclass_fields__:
        cp = dataclasses.replace(cp, needs_layout_passes=False)

    @functools.partial(
        pl.kernel,
        out_type=jax.ShapeDtypeStruct((B * V,), jnp.float32),
        mesh=mesh,
        compiler_params=cp,
        scratch_types=[
            pltpu.VMEM((CH,), jnp.float32),
            pltpu.VMEM((NI,), jnp.int32),
            pltpu.VMEM((NI,), jnp.int32),
        ],
    )
    def k(idx_hbm, off_hbm, out_hbm, buf, idxv, offv):
        wid = lax.axis_index("s") * _NUM_SC_CORES + lax.axis_index("c")
        zeros = jnp.zeros((_SC_LANES,), jnp.float32)
        ones = jnp.ones((_SC_LANES,), jnp.float32)
        lo = jnp.zeros((_SC_LANES,), jnp.int32)
        hi = jnp.full((_SC_LANES,), V - 1, jnp.int32)

        pltpu.sync_copy(idx_hbm.at[pl.ds(wid * NI, NI)], idxv)
        pltpu.sync_copy(off_hbm.at[pl.ds(0, NI)], offv)

        @pl.loop(0, CH, step=_SC_LANES)
        def _(j):
            buf[pl.ds(j, _SC_LANES)] = zeros

        @pl.loop(0, NI, step=_SC_LANES)
        def _(j):
            g = jnp.minimum(jnp.maximum(idxv[pl.ds(j, _SC_LANES)], lo), hi)
            plsc.store_scatter(buf, [g + offv[pl.ds(j, _SC_LANES)]], ones)

        pltpu.sync_copy(buf, out_hbm.at[pl.ds(wid * CH, CH)])

    return k(gs_flat, rowoff)


def _tc_mega(sp, sl, gs, eos_arr, BB):
    B, L, V = sl.shape
    VN = sp.shape[1]
    NVF = VN // V
    inv_v = 1.0 / V

    def body(sp_ref, sl_ref, gs_ref, eos_ref, ent_ref, sle_ref, loss_ref,
             g_ref, mask_ref):
        i = pl.program_id(0)

        @pl.when(i == 0)
        def _():
            r = lax.broadcasted_iota(jnp.int32, (VN, V), 0)
            c = lax.broadcasted_iota(jnp.int32, (VN, V), 1)
            g_ref[...] = ((r // NVF) == c).astype(jnp.bfloat16)

            gsv = gs_ref[...]
            vio = lax.broadcasted_iota(jnp.int32, (B, V), 1)
            pres = (gsv[:, 0:1] == vio)
            for l in range(1, L):
                pres = pres | (gsv[:, l:l + 1] == vio)
            presf = pres.astype(jnp.float32)
            skip = presf.min(axis=0, keepdims=True)
            mask_ref[...] = presf * (1.0 - skip)

        p0 = sp_ref[...]
        t = (p0 * jnp.log(p0)).astype(jnp.bfloat16)
        ent_ref[...] = -lax.dot_general(
            t, g_ref[...], (((1,), (0,)), ((), ())),
            preferred_element_type=jnp.float32)

        x = sl_ref[...]
        mn = x.min(axis=2, keepdims=True)
        em = (lax.broadcasted_iota(jnp.int32, (1, 1, V), 2) == eos_ref[0])
        xs = jnp.where(em, mn, x)
        m = xs.max(axis=1)
        mx = m.max(axis=1, keepdims=True)
        z = m - mx
        e = jnp.exp(z)
        s = e.sum(axis=1, keepdims=True)
        p = e / s
        logp = z - jnp.log(s)
        sle_ref[...] = -(p * logp).sum(axis=1, keepdims=True)
        u = 1.0 - p
        mask = mask_ref[pl.ds(i * BB, BB), :]
        loss_ref[...] = (mask * (u * u)).sum(axis=1, keepdims=True) * inv_v

    return pl.pallas_call(
        body,
        grid=(B // BB,),
        in_specs=[
            pl.BlockSpec((BB, VN), lambda i: (i, 0)),
            pl.BlockSpec((BB, L, V), lambda i: (i, 0, 0)),
            pl.BlockSpec((B, L), lambda i: (0, 0)),
            pl.BlockSpec(memory_space=pltpu.SMEM),
        ],
        out_specs=[
            pl.BlockSpec((BB, V), lambda i: (i, 0)),
            pl.BlockSpec((BB, 1), lambda i: (i, 0)),
            pl.BlockSpec((BB, 1), lambda i: (i, 0)),
        ],
        out_shape=[
            jax.ShapeDtypeStruct((B, V), jnp.float32),
            jax.ShapeDtypeStruct((B, 1), jnp.float32),
            jax.ShapeDtypeStruct((B, 1), jnp.float32),
        ],
        scratch_shapes=[
            pltpu.VMEM((VN, V), jnp.bfloat16),
            pltpu.VMEM((B, V), jnp.float32),
        ],
    )(sp, sl, gs, eos_arr)


def kernel(sentences_logits, visual_features, text_features, semantic_prior,
           semantic_prior_logits, grounding_signal, eos_idx):
    B, L, V = sentences_logits.shape
    ntf = text_features.shape[1]

    gs = grounding_signal.reshape(B, L)
    sp_flat = semantic_prior.reshape(B, -1)
    eos_arr = jnp.asarray(eos_idx, jnp.int32).reshape(1)
    entropy, sle, sentences_loss = _tc_mega(sp_flat, sentences_logits, gs,
                                            eos_arr, BB=256)

    loss = jnp.zeros((B, ntf), jnp.float32)
    return (loss, sentences_loss.reshape(B), entropy, sle.reshape(B))


# speedup vs baseline: 1.0223x; 1.0223x over previous
import dataclasses
import functools

import jax
import jax.numpy as jnp
from jax import lax
from jax.experimental import pallas as pl
from jax.experimental.pallas import tpu as pltpu
from jax.experimental.pallas import tpu_sc as plsc

_NUM_SC_CORES = 2
_NUM_SC_SUBCORES = 16
_SC_LANES = 16


def _sc_present(gs_flat, rowoff, zeros_c, B, V, L):
    NW = _NUM_SC_CORES * _NUM_SC_SUBCORES
    RP = B // NW
    CH = RP * V
    NI = RP * L
    mesh = plsc.VectorSubcoreMesh(core_axis_name="c", subcore_axis_name="s")
    cp = pltpu.CompilerParams()
    if "needs_layout_passes" in pltpu.CompilerParams.__dataclass_fields__:
        cp = dataclasses.replace(cp, needs_layout_passes=False)

    @functools.partial(
        pl.kernel,
        out_type=jax.ShapeDtypeStruct((B * V,), jnp.float32),
        mesh=mesh,
        compiler_params=cp,
        scratch_types=[
            pltpu.VMEM((CH,), jnp.float32),
            pltpu.VMEM((NI,), jnp.int32),
            pltpu.VMEM((NI,), jnp.int32),
        ],
    )
    def k(idx_hbm, off_hbm, z_hbm, out_hbm, buf, idxv, offv):
        wid = lax.axis_index("s") * _NUM_SC_CORES + lax.axis_index("c")
        ones = jnp.ones((_SC_LANES,), jnp.float32)
        lo = jnp.zeros((_SC_LANES,), jnp.int32)
        hi = jnp.full((_SC_LANES,), V - 1, jnp.int32)

        pltpu.sync_copy(idx_hbm.at[pl.ds(wid * NI, NI)], idxv)
        pltpu.sync_copy(off_hbm.at[pl.ds(0, NI)], offv)
        pltpu.sync_copy(z_hbm.at[pl.ds(0, CH)], buf)

        @pl.loop(0, NI, step=_SC_LANES)
        def _(j):
            g = jnp.minimum(jnp.maximum(idxv[pl.ds(j, _SC_LANES)], lo), hi)
            plsc.store_scatter(buf, [g + offv[pl.ds(j, _SC_LANES)]], ones)

        pltpu.sync_copy(buf, out_hbm.at[pl.ds(wid * CH, CH)])

    return k(gs_flat, rowoff, zeros_c)


def _tc_mega(sp, sl, present, eos_arr, BB):
    B, L, V = sl.shape
    VN = sp.shape[1]
    NVF = VN // V
    inv_v = 1.0 / V

    def body(sp_ref, sl_ref, pr_ref, eos_ref, ent_ref, sle_ref, loss_ref,
             g_ref, skip_ref):
        i = pl.program_id(0)

        @pl.when(i == 0)
        def _():
            r = lax.broadcasted_iota(jnp.int32, (VN, V), 0)
            c = lax.broadcasted_iota(jnp.int32, (VN, V), 1)
            g_ref[...] = ((r // NVF) == c).astype(jnp.bfloat16)
            skip_ref[...] = pr_ref[...].min(axis=0, keepdims=True)

        p0 = sp_ref[...]
        t = (p0 * jnp.log(p0)).astype(jnp.bfloat16)
        ent_ref[...] = -lax.dot_general(
            t, g_ref[...], (((1,), (0,)), ((), ())),
            preferred_element_type=jnp.float32)

        x = sl_ref[...]
        mn = x.min(axis=2, keepdims=True)
        em = (lax.broadcasted_iota(jnp.int32, (1, 1, V), 2) == eos_ref[0])
        xs = jnp.where(em, mn, x)
        m = xs.max(axis=1)
        mx = m.max(axis=1, keepdims=True)
        z = m - mx
        e = jnp.exp(z)
        s = e.sum(axis=1, keepdims=True)
        p = e / s
        logp = z - jnp.log(s)
        sle_ref[...] = -(p * logp).sum(axis=1, keepdims=True)
        u = 1.0 - p
        mask = pr_ref[pl.ds(i * BB, BB), :] * (1.0 - skip_ref[...])
        loss_ref[...] = (mask * (u * u)).sum(axis=1, keepdims=True) * inv_v

    return pl.pallas_call(
        body,
        grid=(B // BB,),
        in_specs=[
            pl.BlockSpec((BB, VN), lambda i: (i, 0)),
            pl.BlockSpec((BB, L, V), lambda i: (i, 0, 0)),
            pl.BlockSpec((B, V), lambda i: (0, 0)),
            pl.BlockSpec(memory_space=pltpu.SMEM),
        ],
        out_specs=[
            pl.BlockSpec((BB, V), lambda i: (i, 0)),
            pl.BlockSpec((BB, 1), lambda i: (i, 0)),
            pl.BlockSpec((BB, 1), lambda i: (i, 0)),
        ],
        out_shape=[
            jax.ShapeDtypeStruct((B, V), jnp.float32),
            jax.ShapeDtypeStruct((B, 1), jnp.float32),
            jax.ShapeDtypeStruct((B, 1), jnp.float32),
        ],
        scratch_shapes=[
            pltpu.VMEM((VN, V), jnp.bfloat16),
            pltpu.VMEM((1, V), jnp.float32),
        ],
    )(sp, sl, present, eos_arr)


def kernel(sentences_logits, visual_features, text_features, semantic_prior,
           semantic_prior_logits, grounding_signal, eos_idx):
    B, L, V = sentences_logits.shape
    ntf = text_features.shape[1]

    NW = _NUM_SC_CORES * _NUM_SC_SUBCORES
    RP = B // NW
    gs_flat = grounding_signal.reshape(B * L)
    rowoff = jnp.repeat(jnp.arange(RP, dtype=jnp.int32) * ntf, L)
    zeros_c = jnp.zeros((RP * ntf,), jnp.float32)

    present = _sc_present(gs_flat, rowoff, zeros_c, B, ntf, L).reshape(B, ntf)

    sp_flat = semantic_prior.reshape(B, -1)
    eos_arr = jnp.asarray(eos_idx, jnp.int32).reshape(1)
    entropy, sle, sentences_loss = _tc_mega(sp_flat, sentences_logits,
                                            present, eos_arr, BB=256)

    loss = jnp.zeros((B, ntf), jnp.float32)
    return (loss, sentences_loss.reshape(B), entropy, sle.reshape(B))


# speedup vs baseline: 1.0594x; 1.0362x over previous
import dataclasses
import functools

import jax
import jax.numpy as jnp
from jax import lax
from jax.experimental import pallas as pl
from jax.experimental.pallas import tpu as pltpu
from jax.experimental.pallas import tpu_sc as plsc

_NUM_SC_CORES = 2
_NUM_SC_SUBCORES = 16
_SC_LANES = 16


def _sc_present(gs_flat, rowoff, zeros_c, B, V, L):
    NW = _NUM_SC_CORES * _NUM_SC_SUBCORES
    RP = B // NW
    CH = RP * V
    NI = RP * L
    mesh = plsc.VectorSubcoreMesh(core_axis_name="c", subcore_axis_name="s")
    cp = pltpu.CompilerParams()
    if "needs_layout_passes" in pltpu.CompilerParams.__dataclass_fields__:
        cp = dataclasses.replace(cp, needs_layout_passes=False)

    @functools.partial(
        pl.kernel,
        out_type=jax.ShapeDtypeStruct((B * V,), jnp.float32),
        mesh=mesh,
        compiler_params=cp,
        scratch_types=[
            pltpu.VMEM((CH,), jnp.float32),
            pltpu.VMEM((NI,), jnp.int32),
            pltpu.VMEM((NI,), jnp.int32),
        ],
    )
    def k(idx_hbm, off_hbm, z_hbm, out_hbm, buf, idxv, offv):
        wid = lax.axis_index("s") * _NUM_SC_CORES + lax.axis_index("c")
        ones = jnp.ones((_SC_LANES,), jnp.float32)
        lo = jnp.zeros((_SC_LANES,), jnp.int32)
        hi = jnp.full((_SC_LANES,), V - 1, jnp.int32)

        pltpu.sync_copy(idx_hbm.at[pl.ds(wid * NI, NI)], idxv)
        pltpu.sync_copy(off_hbm.at[pl.ds(0, NI)], offv)
        pltpu.sync_copy(z_hbm.at[pl.ds(0, CH)], buf)

        @pl.loop(0, NI, step=_SC_LANES)
        def _(j):
            g = jnp.minimum(jnp.maximum(idxv[pl.ds(j, _SC_LANES)], lo), hi)
            plsc.store_scatter(buf, [g + offv[pl.ds(j, _SC_LANES)]], ones)

        pltpu.sync_copy(buf, out_hbm.at[pl.ds(wid * CH, CH)])

    return k(gs_flat, rowoff, zeros_c)


def _tc_mega(sp, sl, present, eos_arr, BB):
    B, L, V = sl.shape
    VN = sp.shape[1]
    NVF = VN // V
    inv_v = 1.0 / V

    def body(sp_ref, sl_ref, pr_ref, eos_ref, ent_ref, sle_ref, loss_ref,
             g_ref, skip_ref):
        i = pl.program_id(0)

        @pl.when(i == 0)
        def _():
            r = lax.broadcasted_iota(jnp.int32, (VN, V), 0)
            c = lax.broadcasted_iota(jnp.int32, (VN, V), 1)
            g_ref[...] = ((r // NVF) == c).astype(jnp.bfloat16)
            skip_ref[...] = pr_ref[...].min(axis=0, keepdims=True)

        p0 = sp_ref[...]
        t = (p0 * jnp.log(p0)).astype(jnp.bfloat16)
        ent_ref[...] = -lax.dot_general(
            t, g_ref[...], (((1,), (0,)), ((), ())),
            preferred_element_type=jnp.float32)

        x = sl_ref[...]
        mn = x.min(axis=2, keepdims=True)
        em = (lax.broadcasted_iota(jnp.int32, (1, 1, V), 2) == eos_ref[0])
        xs = jnp.where(em, mn, x)
        m = xs.max(axis=1)
        mx = m.max(axis=1, keepdims=True)
        z = m - mx
        e = jnp.exp(z)
        s = e.sum(axis=1, keepdims=True)
        p = e / s
        logp = z - jnp.log(s)
        sle_ref[...] = (-(p * logp).sum(axis=1, keepdims=True)).reshape(1, BB)
        u = 1.0 - p
        mask = pr_ref[pl.ds(i * BB, BB), :] * (1.0 - skip_ref[...])
        loss_ref[...] = ((mask * (u * u)).sum(axis=1, keepdims=True)
                         * inv_v).reshape(1, BB)

    return pl.pallas_call(
        body,
        grid=(B // BB,),
        in_specs=[
            pl.BlockSpec((BB, VN), lambda i: (i, 0)),
            pl.BlockSpec((BB, L, V), lambda i: (i, 0, 0)),
            pl.BlockSpec((B, V), lambda i: (0, 0)),
            pl.BlockSpec(memory_space=pltpu.SMEM),
        ],
        out_specs=[
            pl.BlockSpec((BB, V), lambda i: (i, 0)),
            pl.BlockSpec((1, BB), lambda i: (0, i)),
            pl.BlockSpec((1, BB), lambda i: (0, i)),
        ],
        out_shape=[
            jax.ShapeDtypeStruct((B, V), jnp.float32),
            jax.ShapeDtypeStruct((1, B), jnp.float32),
            jax.ShapeDtypeStruct((1, B), jnp.float32),
        ],
        scratch_shapes=[
            pltpu.VMEM((VN, V), jnp.bfloat16),
            pltpu.VMEM((1, V), jnp.float32),
        ],
    )(sp, sl, present, eos_arr)


def kernel(sentences_logits, visual_features, text_features, semantic_prior,
           semantic_prior_logits, grounding_signal, eos_idx):
    B, L, V = sentences_logits.shape
    ntf = text_features.shape[1]

    NW = _NUM_SC_CORES * _NUM_SC_SUBCORES
    RP = B // NW
    gs_flat = grounding_signal.reshape(B * L)
    rowoff = jnp.repeat(jnp.arange(RP, dtype=jnp.int32) * ntf, L)
    zeros_c = jnp.zeros((RP * ntf,), jnp.float32)

    present = _sc_present(gs_flat, rowoff, zeros_c, B, ntf, L).reshape(B, ntf)

    sp_flat = semantic_prior.reshape(B, -1)
    eos_arr = jnp.asarray(eos_idx, jnp.int32).reshape(1)
    entropy, sle, sentences_loss = _tc_mega(sp_flat, sentences_logits,
                                            present, eos_arr, BB=256)

    loss = jnp.zeros((B, ntf), jnp.float32)
    return (loss, sentences_loss.reshape(B), entropy, sle.reshape(B))
